# baseline (speedup 1.0000x reference)
import functools

import jax
import jax.numpy as jnp
import numpy as np
from jax import lax
from jax.experimental import pallas as pl
from jax.experimental.pallas import tpu as pltpu
from jax.experimental.pallas import tpu_sc as plsc

N = 10000
E = 320000
D_IN = 128
H = 64
EPS = 1e-5

NC = 2
NS = 16
NW = NC * NS
CHUNK = 128
NBUF = 4
LEAD = NBUF - 2
N_PAD = 10240
ROWS_PER_SUB = N_PAD // NS

CPT = -(-E // (NW * CHUNK))
EPT = CPT * CHUNK
E_PAD = EPT * NW

_SC_MESH = plsc.VectorSubcoreMesh(core_axis_name="c", subcore_axis_name="s")
_SC_PARAMS = pltpu.CompilerParams(use_tc_tiling_on_sc=False,
                                  needs_layout_passes=False)

_PERM = np.array([blk * 32 + (k // 2) + 16 * (k % 2)
                  for blk in range(H // 32) for k in range(32)])


def _zero_rows(zbuf, acc, base, rows):
    def zb(i, _):
        for f in range(H // 16):
            zbuf[i, pl.ds(f * 16, 16)] = jnp.zeros((16,), jnp.float32)
        return 0
    lax.fori_loop(0, CHUNK, zb, 0, unroll=4)
    for k in range(rows // CHUNK):
        pltpu.sync_copy(zbuf, acc.at[pl.ds(base + k * CHUNK, CHUNK)])


@functools.partial(
    pl.kernel,
    out_type=jax.ShapeDtypeStruct((NC, N_PAD), jnp.float32),
    mesh=_SC_MESH,
    scratch_types=[
        pltpu.VMEM_SHARED((N_PAD,), jnp.float32),
        pltpu.VMEM((CPT, CHUNK), jnp.int32),
        pltpu.VMEM((CPT, CHUNK), jnp.float32),
        pltpu.VMEM((CHUNK,), jnp.float32),
        pltpu.SemaphoreType.DMA,
    ],
)
def _deg_kernel(col_hbm, ew_hbm, out_hbm, acc, col_t, ew_t, zbuf, sem):
    c = lax.axis_index("c")
    s = lax.axis_index("s")
    wid = c * NS + s

    def zb(i, _):
        zbuf[pl.ds(i * 16, 16)] = jnp.zeros((16,), jnp.float32)
        return 0
    lax.fori_loop(0, CHUNK // 16, zb, 0, unroll=4)
    for k in range(ROWS_PER_SUB // CHUNK):
        pltpu.sync_copy(zbuf, acc.at[pl.ds(s * ROWS_PER_SUB + k * CHUNK,
                                           CHUNK)])
    pltpu.sync_copy(col_hbm.at[wid], col_t)
    pltpu.sync_copy(ew_hbm.at[wid], ew_t)
    plsc.subcore_barrier()

    def fire(k, _):
        pltpu.make_async_copy(ew_t.at[k], acc.at[col_t.at[k]], sem).start(
            add=True)
        return 0

    lax.fori_loop(0, CPT, fire, 0)

    def drain(k, _):
        pltpu.make_async_copy(ew_t.at[0], acc.at[col_t.at[0]], sem).wait()
        return 0

    lax.fori_loop(0, CPT, drain, 0)
    plsc.subcore_barrier()
    pltpu.sync_copy(
        acc.at[pl.ds(s * ROWS_PER_SUB, ROWS_PER_SUB)],
        out_hbm.at[c, pl.ds(s * ROWS_PER_SUB, ROWS_PER_SUB)],
    )


@functools.partial(
    pl.kernel,
    out_type=jax.ShapeDtypeStruct((NC, N, H), jnp.float32),
    mesh=_SC_MESH,
    scratch_types=[
        pltpu.VMEM_SHARED((N, H), jnp.float32),
        pltpu.VMEM((CPT, CHUNK), jnp.int32),
        pltpu.VMEM((CPT, CHUNK), jnp.int32),
        pltpu.VMEM((CPT, CHUNK), jnp.float32),
        pltpu.VMEM((NBUF, CHUNK, H), jnp.bfloat16),
        pltpu.VMEM((NBUF, CHUNK, H), jnp.float32),
        pltpu.SemaphoreType.DMA((NBUF,)),
        pltpu.SemaphoreType.DMA((NBUF,)),
    ],
    compiler_params=_SC_PARAMS,
)
def _agg_kernel(y_hbm, row_hbm, col_hbm, ew_hbm, out_hbm,
                acc, row_t, col_t, ew_t, rows, msg, gsem, ssem):
    c = lax.axis_index("c")
    s = lax.axis_index("s")
    wid = c * NS + s

    def zb(i, _):
        for f in range(H // 16):
            msg[0, i, pl.ds(f * 16, 16)] = jnp.zeros((16,), jnp.float32)
        return 0
    lax.fori_loop(0, 125, zb, 0, unroll=4)
    for k in range(5):
        pltpu.sync_copy(msg.at[0, pl.ds(0, 125)],
                        acc.at[pl.ds(s * 625 + k * 125, 125)])
    pltpu.sync_copy(row_hbm.at[wid], row_t)
    pltpu.sync_copy(col_hbm.at[wid], col_t)
    pltpu.sync_copy(ew_hbm.at[wid], ew_t)
    plsc.subcore_barrier()

    def gather(k, b):
        return pltpu.make_async_copy(y_hbm.at[row_t.at[k]], rows.at[b],
                                     gsem.at[b])

    def scatter(k, b):
        return pltpu.make_async_copy(msg.at[b], acc.at[col_t.at[k]],
                                     ssem.at[b])

    for b in range(LEAD):
        gather(b, b).start()

    def step(k, b):
        gather(k, b).wait()

        kv = jnp.full((16,), k, jnp.int32)

        def scale(g, _):
            gv = jnp.full((16,), g * 16, jnp.int32)
            for j in range(16):
                w = plsc.load_gather(ew_t, [kv, gv + j])
                e = g * 16 + j
                for blk in range(H // 32):
                    v = rows[b, e, pl.ds(blk * 32, 32)]
                    lo, hi = plsc.unpack(
                        v, format=plsc.PackFormat.INTERLEAVED)
                    msg[b, e, pl.ds(blk * 32, 16)] = lo * w
                    msg[b, e, pl.ds(blk * 32 + 16, 16)] = hi * w
            return 0

        lax.fori_loop(0, CHUNK // 16, scale, 0)
        scatter(k, b).start(add=True)
        kn = k + LEAD
        bn = (b + LEAD) % NBUF

        @pl.when((kn < CPT) & (k >= NBUF - LEAD))
        def _():
            scatter(0, bn).wait()

        @pl.when(kn < CPT)
        def _():
            gather(kn, bn).start()

    def body(kk, _):
        for b in range(NBUF):
            step(kk * NBUF + b, b)
        return 0

    lax.fori_loop(0, CPT // NBUF, body, 0)
    for b in range(CPT % NBUF):
        step((CPT // NBUF) * NBUF + b, b)

    for b in range(NBUF):
        scatter(0, b).wait()

    plsc.subcore_barrier()
    pltpu.sync_copy(
        acc.at[pl.ds(s * 625, 625)],
        out_hbm.at[c, pl.ds(s * 625, 625)],
    )


def _bn(h, g, be):
    mean = jnp.mean(h, axis=0, keepdims=True)
    var = jnp.mean((h - mean) ** 2, axis=0, keepdims=True)
    return (h - mean) * lax.rsqrt(var + EPS) * g + be


def _pre_body(x_ref, w1_ref, w1p_ref, degp_ref, dinv_ref, y1_ref, y1m_ref):
    deg = degp_ref[0, :N] + degp_ref[1, :N] + 1.0
    dinv = lax.rsqrt(deg)
    dinv_ref[...] = dinv
    x = x_ref[...]
    y1_ref[...] = jnp.dot(x, w1_ref[...],
                          preferred_element_type=jnp.float32) * dinv
    y1m_ref[...] = (jnp.dot(x, w1p_ref[...],
                            preferred_element_type=jnp.float32)
                    * dinv).astype(jnp.bfloat16)


def _mid_body(aggp_ref, y1_ref, dinv_ref, w2_ref, w2p_ref, b1_ref, g1_ref,
              be1_ref, y2_ref, y2m_ref):
    dinv = dinv_ref[...]
    agg = aggp_ref[0] + aggp_ref[1]
    h = dinv * (agg + y1_ref[...]) + b1_ref[...]
    h = jnp.maximum(h, 0.0)
    t = _bn(h, g1_ref[...], be1_ref[...])
    y2_ref[...] = jnp.dot(t, w2_ref[...],
                          preferred_element_type=jnp.float32) * dinv
    y2m_ref[...] = (jnp.dot(t, w2p_ref[...],
                            preferred_element_type=jnp.float32)
                    * dinv).astype(jnp.bfloat16)


def _post_body(aggp_ref, y2_ref, dinv_ref, b2_ref, g2_ref, be2_ref,
               wl1_ref, bl1_ref, g3_ref, be3_ref, wl2_ref, bl2_ref, out_ref):
    dinv = dinv_ref[...]
    agg = aggp_ref[0] + aggp_ref[1]
    h = dinv * (agg + y2_ref[...]) + b2_ref[...]
    t = _bn(h, g2_ref[...], be2_ref[...])
    h3 = jnp.dot(t, wl1_ref[...], preferred_element_type=jnp.float32) \
        + bl1_ref[...]
    t3 = _bn(h3, g3_ref[...], be3_ref[...])
    out_ref[...] = jnp.dot(t3, wl2_ref[...],
                           preferred_element_type=jnp.float32) + bl2_ref[...]


def _tc_call(body, out_shapes, *args):
    return pl.pallas_call(body, out_shape=out_shapes)(*args)


def kernel(x, edge_index, edge_attr, W1, b1, g1, be1, W2, b2, g2, be2,
           Wl1, bl1, g3, be3, Wl2, bl2):
    row = edge_index[0].astype(jnp.int32)
    col = edge_index[1].astype(jnp.int32)
    ew = edge_attr.astype(jnp.float32)
    pad = E_PAD - E
    row_p = jnp.concatenate([row, jnp.zeros((pad,), jnp.int32)])
    col_p = jnp.concatenate([col, jnp.zeros((pad,), jnp.int32)])
    ew_p = jnp.concatenate([ew, jnp.zeros((pad,), jnp.float32)])
    row3 = row_p.reshape(NW, CPT, CHUNK)
    col3 = col_p.reshape(NW, CPT, CHUNK)
    ew3 = ew_p.reshape(NW, CPT, CHUNK)

    degp = _deg_kernel(col3, ew3)

    W1p = W1[:, _PERM]
    W2p = W2[:, _PERM]
    dinv, y1, y1m = _tc_call(
        _pre_body,
        (jax.ShapeDtypeStruct((N, 1), jnp.float32),
         jax.ShapeDtypeStruct((N, H), jnp.float32),
         jax.ShapeDtypeStruct((N, H), jnp.bfloat16)),
        x, W1, W1p, degp.reshape(NC, N_PAD, 1))

    agg1 = _agg_kernel(y1m, row3, col3, ew3)

    y2, y2m = _tc_call(
        _mid_body,
        (jax.ShapeDtypeStruct((N, H), jnp.float32),
         jax.ShapeDtypeStruct((N, H), jnp.bfloat16)),
        agg1, y1, dinv, W2, W2p,
        b1.reshape(1, H), g1.reshape(1, H), be1.reshape(1, H))

    agg2 = _agg_kernel(y2m, row3, col3, ew3)

    out = _tc_call(
        _post_body,
        jax.ShapeDtypeStruct((N, 1), jnp.float32),
        agg2, y2, dinv,
        b2.reshape(1, H), g2.reshape(1, H), be2.reshape(1, H),
        Wl1, bl1.reshape(1, H), g3.reshape(1, H), be3.reshape(1, H),
        Wl2, bl2.reshape(1, 1))

    return out.reshape(N)


# speedup vs baseline: 1.0002x; 1.0002x over previous
import functools

import jax
import jax.numpy as jnp
import numpy as np
from jax import lax
from jax.experimental import pallas as pl
from jax.experimental.pallas import tpu as pltpu
from jax.experimental.pallas import tpu_sc as plsc

N = 10000
E = 320000
D_IN = 128
H = 64
EPS = 1e-5

NC = 2
NS = 16
NW = NC * NS
CHUNK = 128
NBUF = 4
LEAD = NBUF - 2
N_PAD = 10240
ROWS_PER_SUB = N_PAD // NS

CPT = -(-E // (NW * CHUNK))
EPT = CPT * CHUNK
E_PAD = EPT * NW

_SC_MESH = plsc.VectorSubcoreMesh(core_axis_name="c", subcore_axis_name="s")
_SC_PARAMS = pltpu.CompilerParams(use_tc_tiling_on_sc=False,
                                  needs_layout_passes=False)

_PERM = np.array([blk * 32 + (k // 2) + 16 * (k % 2)
                  for blk in range(H // 32) for k in range(32)])


def _zero_rows(zbuf, acc, base, rows):
    def zb(i, _):
        for f in range(H // 16):
            zbuf[i, pl.ds(f * 16, 16)] = jnp.zeros((16,), jnp.float32)
        return 0
    lax.fori_loop(0, CHUNK, zb, 0, unroll=4)
    for k in range(rows // CHUNK):
        pltpu.sync_copy(zbuf, acc.at[pl.ds(base + k * CHUNK, CHUNK)])


@functools.partial(
    pl.kernel,
    out_type=jax.ShapeDtypeStruct((NC, N_PAD), jnp.float32),
    mesh=_SC_MESH,
    scratch_types=[
        pltpu.VMEM_SHARED((N_PAD,), jnp.float32),
        pltpu.VMEM((CPT, CHUNK), jnp.int32),
        pltpu.VMEM((CPT, CHUNK), jnp.float32),
        pltpu.VMEM((CHUNK,), jnp.float32),
        pltpu.SemaphoreType.DMA,
    ],
)
def _deg_kernel(col_hbm, ew_hbm, out_hbm, acc, col_t, ew_t, zbuf, sem):
    c = lax.axis_index("c")
    s = lax.axis_index("s")
    wid = c * NS + s

    def zb(i, _):
        zbuf[pl.ds(i * 16, 16)] = jnp.zeros((16,), jnp.float32)
        return 0
    lax.fori_loop(0, CHUNK // 16, zb, 0, unroll=4)
    for k in range(ROWS_PER_SUB // CHUNK):
        pltpu.sync_copy(zbuf, acc.at[pl.ds(s * ROWS_PER_SUB + k * CHUNK,
                                           CHUNK)])
    pltpu.sync_copy(col_hbm.at[wid], col_t)
    pltpu.sync_copy(ew_hbm.at[wid], ew_t)
    plsc.subcore_barrier()

    def fire(k, _):
        pltpu.make_async_copy(ew_t.at[k], acc.at[col_t.at[k]], sem).start(
            add=True)
        return 0

    lax.fori_loop(0, CPT, fire, 0)

    def drain(k, _):
        pltpu.make_async_copy(ew_t.at[0], acc.at[col_t.at[0]], sem).wait()
        return 0

    lax.fori_loop(0, CPT, drain, 0)
    plsc.subcore_barrier()
    pltpu.sync_copy(
        acc.at[pl.ds(s * ROWS_PER_SUB, ROWS_PER_SUB)],
        out_hbm.at[c, pl.ds(s * ROWS_PER_SUB, ROWS_PER_SUB)],
    )


@functools.partial(
    pl.kernel,
    out_type=jax.ShapeDtypeStruct((NC, N, H), jnp.bfloat16),
    mesh=_SC_MESH,
    scratch_types=[
        pltpu.VMEM_SHARED((N, H), jnp.float32),
        pltpu.VMEM((CPT, CHUNK), jnp.int32),
        pltpu.VMEM((CPT, CHUNK), jnp.int32),
        pltpu.VMEM((CPT, CHUNK), jnp.float32),
        pltpu.VMEM((NBUF, CHUNK, H), jnp.bfloat16),
        pltpu.VMEM((NBUF, CHUNK, H), jnp.float32),
        pltpu.SemaphoreType.DMA((NBUF,)),
        pltpu.SemaphoreType.DMA((NBUF,)),
    ],
    compiler_params=_SC_PARAMS,
)
def _agg_kernel(y_hbm, row_hbm, col_hbm, ew_hbm, out_hbm,
                acc, row_t, col_t, ew_t, rows, msg, gsem, ssem):
    c = lax.axis_index("c")
    s = lax.axis_index("s")
    wid = c * NS + s

    def zb(i, _):
        for f in range(H // 16):
            msg[0, i, pl.ds(f * 16, 16)] = jnp.zeros((16,), jnp.float32)
        return 0
    lax.fori_loop(0, 125, zb, 0, unroll=4)
    for k in range(5):
        pltpu.sync_copy(msg.at[0, pl.ds(0, 125)],
                        acc.at[pl.ds(s * 625 + k * 125, 125)])
    pltpu.sync_copy(row_hbm.at[wid], row_t)
    pltpu.sync_copy(col_hbm.at[wid], col_t)
    pltpu.sync_copy(ew_hbm.at[wid], ew_t)
    plsc.subcore_barrier()

    def gather(k, b):
        return pltpu.make_async_copy(y_hbm.at[row_t.at[k]], rows.at[b],
                                     gsem.at[b])

    def scatter(k, b):
        return pltpu.make_async_copy(msg.at[b], acc.at[col_t.at[k]],
                                     ssem.at[b])

    for b in range(LEAD):
        gather(b, b).start()

    def step(k, b):
        gather(k, b).wait()

        kv = jnp.full((16,), k, jnp.int32)

        def scale(g, _):
            gv = jnp.full((16,), g * 16, jnp.int32)
            for j in range(16):
                w = plsc.load_gather(ew_t, [kv, gv + j])
                e = g * 16 + j
                for blk in range(H // 32):
                    v = rows[b, e, pl.ds(blk * 32, 32)]
                    lo, hi = plsc.unpack(
                        v, format=plsc.PackFormat.INTERLEAVED)
                    msg[b, e, pl.ds(blk * 32, 16)] = lo * w
                    msg[b, e, pl.ds(blk * 32 + 16, 16)] = hi * w
            return 0

        lax.fori_loop(0, CHUNK // 16, scale, 0)
        scatter(k, b).start(add=True)
        kn = k + LEAD
        bn = (b + LEAD) % NBUF

        @pl.when((kn < CPT) & (k >= NBUF - LEAD))
        def _():
            scatter(0, bn).wait()

        @pl.when(kn < CPT)
        def _():
            gather(kn, bn).start()

    def body(kk, _):
        for b in range(NBUF):
            step(kk * NBUF + b, b)
        return 0

    lax.fori_loop(0, CPT // NBUF, body, 0)
    for b in range(CPT % NBUF):
        step((CPT // NBUF) * NBUF + b, b)

    for b in range(NBUF):
        scatter(0, b).wait()

    plsc.subcore_barrier()
    for blk in range(5):
        base = s * 625 + blk * 125
        pltpu.sync_copy(acc.at[pl.ds(base, 125)], msg.at[0, pl.ds(0, 125)])

        def cv(i, _):
            for f2 in range(H // 32):
                lo = msg[0, i, pl.ds(f2 * 32, 16)]
                hi = msg[0, i, pl.ds(f2 * 32 + 16, 16)]
                rows[0, i, pl.ds(f2 * 32, 32)] = plsc.pack(
                    lo, hi, format=plsc.PackFormat.INTERLEAVED)
            return 0

        lax.fori_loop(0, 125, cv, 0, unroll=4)
        pltpu.sync_copy(rows.at[0, pl.ds(0, 125)],
                        out_hbm.at[c, pl.ds(base, 125)])


def _bn(h, g, be):
    mean = jnp.mean(h, axis=0, keepdims=True)
    var = jnp.mean(h * h, axis=0, keepdims=True) - mean * mean
    return (h - mean) * lax.rsqrt(var + EPS) * g + be


def _pre_body(x_ref, w1_ref, w1p_ref, degp_ref, dinv_ref, y1_ref, y1m_ref):
    deg_row = degp_ref[0:1, :N] + degp_ref[1:2, :N] + 1.0
    dinv_row = lax.rsqrt(deg_row)
    dinv = lax.dot_general(dinv_row, jnp.ones((1, 1), jnp.float32),
                           (((0,), (0,)), ((), ())),
                           preferred_element_type=jnp.float32)
    dinv_ref[...] = dinv
    x = x_ref[...]
    y1_ref[...] = jnp.dot(x, w1_ref[...],
                          preferred_element_type=jnp.float32) * dinv
    y1m_ref[...] = (jnp.dot(x, w1p_ref[...],
                            preferred_element_type=jnp.float32)
                    * dinv).astype(jnp.bfloat16)


def _mid_body(aggp_ref, y1_ref, dinv_ref, w2_ref, w2p_ref, p_ref, b1_ref,
              g1_ref, be1_ref, y2_ref, y2m_ref):
    dinv = dinv_ref[...]
    aggs = (aggp_ref[0].astype(jnp.float32)
            + aggp_ref[1].astype(jnp.float32))
    agg = jnp.dot(aggs, p_ref[...], preferred_element_type=jnp.float32)
    h = dinv * (agg + y1_ref[...]) + b1_ref[...]
    h = jnp.maximum(h, 0.0)
    t = _bn(h, g1_ref[...], be1_ref[...])
    y2_ref[...] = jnp.dot(t, w2_ref[...],
                          preferred_element_type=jnp.float32) * dinv
    y2m_ref[...] = (jnp.dot(t, w2p_ref[...],
                            preferred_element_type=jnp.float32)
                    * dinv).astype(jnp.bfloat16)


def _post_body(aggp_ref, y2_ref, dinv_ref, p_ref, b2_ref, g2_ref, be2_ref,
               wl1_ref, bl1_ref, g3_ref, be3_ref, wl2_ref, bl2_ref, out_ref):
    dinv = dinv_ref[...]
    aggs = (aggp_ref[0].astype(jnp.float32)
            + aggp_ref[1].astype(jnp.float32))
    agg = jnp.dot(aggs, p_ref[...], preferred_element_type=jnp.float32)
    h = dinv * (agg + y2_ref[...]) + b2_ref[...]
    t = _bn(h, g2_ref[...], be2_ref[...])
    h3 = jnp.dot(t, wl1_ref[...], preferred_element_type=jnp.float32) \
        + bl1_ref[...]
    t3 = _bn(h3, g3_ref[...], be3_ref[...])
    out_ref[...] = jnp.dot(t3, wl2_ref[...],
                           preferred_element_type=jnp.float32) + bl2_ref[...]


def _tc_call(body, out_shapes, *args):
    return pl.pallas_call(body, out_shape=out_shapes)(*args)


def kernel(x, edge_index, edge_attr, W1, b1, g1, be1, W2, b2, g2, be2,
           Wl1, bl1, g3, be3, Wl2, bl2):
    row = edge_index[0].astype(jnp.int32)
    col = edge_index[1].astype(jnp.int32)
    ew = edge_attr.astype(jnp.float32)
    pad = E_PAD - E
    row_p = jnp.concatenate([row, jnp.zeros((pad,), jnp.int32)])
    col_p = jnp.concatenate([col, jnp.zeros((pad,), jnp.int32)])
    ew_p = jnp.concatenate([ew, jnp.zeros((pad,), jnp.float32)])
    row3 = row_p.reshape(NW, CPT, CHUNK)
    col3 = col_p.reshape(NW, CPT, CHUNK)
    ew3 = ew_p.reshape(NW, CPT, CHUNK)

    degp = _deg_kernel(col3, ew3)

    W1p = W1[:, _PERM]
    W2p = W2[:, _PERM]
    P = jnp.eye(H, dtype=jnp.float32)[_PERM]
    dinv, y1, y1m = _tc_call(
        _pre_body,
        (jax.ShapeDtypeStruct((N, 1), jnp.float32),
         jax.ShapeDtypeStruct((N, H), jnp.float32),
         jax.ShapeDtypeStruct((N, H), jnp.bfloat16)),
        x, W1, W1p, degp)

    agg1 = _agg_kernel(y1m, row3, col3, ew3)

    y2, y2m = _tc_call(
        _mid_body,
        (jax.ShapeDtypeStruct((N, H), jnp.float32),
         jax.ShapeDtypeStruct((N, H), jnp.bfloat16)),
        agg1, y1, dinv, W2, W2p, P,
        b1.reshape(1, H), g1.reshape(1, H), be1.reshape(1, H))

    agg2 = _agg_kernel(y2m, row3, col3, ew3)

    out = _tc_call(
        _post_body,
        jax.ShapeDtypeStruct((N, 1), jnp.float32),
        agg2, y2, dinv, P,
        b2.reshape(1, H), g2.reshape(1, H), be2.reshape(1, H),
        Wl1, bl1.reshape(1, H), g3.reshape(1, H), be3.reshape(1, H),
        Wl2, bl2.reshape(1, 1))

    return out.reshape(N)


# speedup vs baseline: 1.0659x; 1.0657x over previous
import functools

import jax
import jax.numpy as jnp
import numpy as np
from jax import lax
from jax.experimental import pallas as pl
from jax.experimental.pallas import tpu as pltpu
from jax.experimental.pallas import tpu_sc as plsc

N = 10000
E = 320000
D_IN = 128
H = 64
EPS = 1e-5

NC = 2
NS = 16
NW = NC * NS
CHUNK = 128
NBUF = 4
LEAD = NBUF - 1
N_PAD = 10240
ROWS_PER_SUB = N_PAD // NS

CPT = -(-E // (NW * CHUNK))
EPT = CPT * CHUNK
E_PAD = EPT * NW

_SC_MESH = plsc.VectorSubcoreMesh(core_axis_name="c", subcore_axis_name="s")
_SC_PARAMS = pltpu.CompilerParams(use_tc_tiling_on_sc=False,
                                  needs_layout_passes=False)

_PERM = np.array([blk * 32 + (k // 2) + 16 * (k % 2)
                  for blk in range(H // 32) for k in range(32)])


def _zero_rows(zbuf, acc, base, rows):
    def zb(i, _):
        for f in range(H // 16):
            zbuf[i, pl.ds(f * 16, 16)] = jnp.zeros((16,), jnp.float32)
        return 0
    lax.fori_loop(0, CHUNK, zb, 0, unroll=4)
    for k in range(rows // CHUNK):
        pltpu.sync_copy(zbuf, acc.at[pl.ds(base + k * CHUNK, CHUNK)])


@functools.partial(
    pl.kernel,
    out_type=jax.ShapeDtypeStruct((NC, N_PAD), jnp.float32),
    mesh=_SC_MESH,
    scratch_types=[
        pltpu.VMEM_SHARED((N_PAD,), jnp.float32),
        pltpu.VMEM((CPT, CHUNK), jnp.int32),
        pltpu.VMEM((CPT, CHUNK), jnp.float32),
        pltpu.VMEM((CHUNK,), jnp.float32),
        pltpu.SemaphoreType.DMA,
    ],
)
def _deg_kernel(col_hbm, ew_hbm, out_hbm, acc, col_t, ew_t, zbuf, sem):
    c = lax.axis_index("c")
    s = lax.axis_index("s")
    wid = c * NS + s

    def zb(i, _):
        zbuf[pl.ds(i * 16, 16)] = jnp.zeros((16,), jnp.float32)
        return 0
    lax.fori_loop(0, CHUNK // 16, zb, 0, unroll=4)
    for k in range(ROWS_PER_SUB // CHUNK):
        pltpu.sync_copy(zbuf, acc.at[pl.ds(s * ROWS_PER_SUB + k * CHUNK,
                                           CHUNK)])
    pltpu.sync_copy(col_hbm.at[wid], col_t)
    pltpu.sync_copy(ew_hbm.at[wid], ew_t)
    plsc.subcore_barrier()

    def fire(k, _):
        pltpu.make_async_copy(ew_t.at[k], acc.at[col_t.at[k]], sem).start(
            add=True)
        return 0

    lax.fori_loop(0, CPT, fire, 0)

    def drain(k, _):
        pltpu.make_async_copy(ew_t.at[0], acc.at[col_t.at[0]], sem).wait()
        return 0

    lax.fori_loop(0, CPT, drain, 0)
    plsc.subcore_barrier()
    pltpu.sync_copy(
        acc.at[pl.ds(s * ROWS_PER_SUB, ROWS_PER_SUB)],
        out_hbm.at[c, pl.ds(s * ROWS_PER_SUB, ROWS_PER_SUB)],
    )


@functools.partial(
    pl.kernel,
    out_type=jax.ShapeDtypeStruct((NC, N, H), jnp.float32),
    mesh=_SC_MESH,
    scratch_types=[
        pltpu.VMEM_SHARED((N, H), jnp.float32),
        pltpu.VMEM((CPT, CHUNK), jnp.int32),
        pltpu.VMEM((CPT, CHUNK), jnp.int32),
        pltpu.VMEM((CPT, CHUNK), jnp.float32),
        pltpu.VMEM((NBUF, CHUNK, H), jnp.bfloat16),
        pltpu.VMEM((NBUF, CHUNK, H), jnp.float32),
        pltpu.SemaphoreType.DMA((NBUF,)),
        pltpu.SemaphoreType.DMA((NBUF,)),
    ],
    compiler_params=_SC_PARAMS,
)
def _agg_kernel(y_hbm, row_hbm, col_hbm, ew_hbm, out_hbm,
                acc, row_t, col_t, ew_t, rows, msg, gsem, ssem):
    c = lax.axis_index("c")
    s = lax.axis_index("s")
    wid = c * NS + s

    def zb(i, _):
        for f in range(H // 16):
            msg[0, i, pl.ds(f * 16, 16)] = jnp.zeros((16,), jnp.float32)
        return 0
    lax.fori_loop(0, 125, zb, 0, unroll=4)
    for k in range(5):
        pltpu.sync_copy(msg.at[0, pl.ds(0, 125)],
                        acc.at[pl.ds(s * 625 + k * 125, 125)])
    pltpu.sync_copy(row_hbm.at[wid], row_t)
    pltpu.sync_copy(col_hbm.at[wid], col_t)
    pltpu.sync_copy(ew_hbm.at[wid], ew_t)
    plsc.subcore_barrier()

    def gather(k, b):
        return pltpu.make_async_copy(y_hbm.at[row_t.at[k]], rows.at[b],
                                     gsem.at[b])

    def scatter(k, b):
        return pltpu.make_async_copy(msg.at[b], acc.at[col_t.at[k]],
                                     ssem.at[b])

    for b in range(LEAD):
        gather(b, b).start()

    def step(k, b):
        gather(k, b).wait()

        kv = jnp.full((16,), k, jnp.int32)

        def scale(g, _):
            gv = jnp.full((16,), g * 16, jnp.int32)
            for j in range(16):
                w = plsc.load_gather(ew_t, [kv, gv + j])
                e = g * 16 + j
                for blk in range(H // 32):
                    v = rows[b, e, pl.ds(blk * 32, 32)]
                    lo, hi = plsc.unpack(
                        v, format=plsc.PackFormat.INTERLEAVED)
                    msg[b, e, pl.ds(blk * 32, 16)] = lo * w
                    msg[b, e, pl.ds(blk * 32 + 16, 16)] = hi * w
            return 0

        lax.fori_loop(0, CHUNK // 16, scale, 0)
        scatter(k, b).start(add=True)
        kn = k + LEAD
        bn = (b + LEAD) % NBUF

        @pl.when((kn < CPT) & (k >= NBUF - LEAD))
        def _():
            scatter(0, bn).wait()

        @pl.when(kn < CPT)
        def _():
            gather(kn, bn).start()

    def body(kk, _):
        for b in range(NBUF):
            step(kk * NBUF + b, b)
        return 0

    lax.fori_loop(0, CPT // NBUF, body, 0)
    for b in range(CPT % NBUF):
        step((CPT // NBUF) * NBUF + b, b)

    for b in range(NBUF):
        scatter(0, b).wait()

    plsc.subcore_barrier()
    pltpu.sync_copy(
        acc.at[pl.ds(s * 625, 625)],
        out_hbm.at[c, pl.ds(s * 625, 625)],
    )


def _bn(h, g, be):
    mean = jnp.mean(h, axis=0, keepdims=True)
    var = jnp.mean(h * h, axis=0, keepdims=True) - mean * mean
    return (h - mean) * lax.rsqrt(var + EPS) * g + be


def _pre_body(x_ref, w1_ref, w1p_ref, degp_ref, dinv_ref, y1_ref, y1m_ref):
    deg_row = degp_ref[0:1, :N] + degp_ref[1:2, :N] + 1.0
    dinv_row = lax.rsqrt(deg_row)
    dinv = lax.dot_general(dinv_row, jnp.ones((1, 1), jnp.float32),
                           (((0,), (0,)), ((), ())),
                           preferred_element_type=jnp.float32)
    dinv_ref[...] = dinv
    x = x_ref[...]
    y1_ref[...] = jnp.dot(x, w1_ref[...],
                          preferred_element_type=jnp.float32) * dinv
    y1m_ref[...] = (jnp.dot(x, w1p_ref[...],
                            preferred_element_type=jnp.float32)
                    * dinv).astype(jnp.bfloat16)


def _mid_body(aggp_ref, y1_ref, dinv_ref, w2_ref, w2p_ref, b1_ref, g1_ref,
              be1_ref, y2_ref, y2m_ref):
    dinv = dinv_ref[...]
    agg = aggp_ref[0] + aggp_ref[1]
    h = dinv * (agg + y1_ref[...]) + b1_ref[...]
    h = jnp.maximum(h, 0.0)
    t = _bn(h, g1_ref[...], be1_ref[...])
    y2_ref[...] = jnp.dot(t, w2_ref[...],
                          preferred_element_type=jnp.float32) * dinv
    y2m_ref[...] = (jnp.dot(t, w2p_ref[...],
                            preferred_element_type=jnp.float32)
                    * dinv).astype(jnp.bfloat16)


def _post_body(aggp_ref, y2_ref, dinv_ref, b2_ref, g2_ref, be2_ref,
               wl1_ref, bl1_ref, g3_ref, be3_ref, wl2_ref, bl2_ref, out_ref):
    dinv = dinv_ref[...]
    agg = aggp_ref[0] + aggp_ref[1]
    h = dinv * (agg + y2_ref[...]) + b2_ref[...]
    t = _bn(h, g2_ref[...], be2_ref[...])
    h3 = jnp.dot(t, wl1_ref[...], preferred_element_type=jnp.float32) \
        + bl1_ref[...]
    t3 = _bn(h3, g3_ref[...], be3_ref[...])
    out_ref[...] = jnp.dot(t3, wl2_ref[...],
                           preferred_element_type=jnp.float32) + bl2_ref[...]


def _tc_call(body, out_shapes, *args):
    return pl.pallas_call(body, out_shape=out_shapes)(*args)


def kernel(x, edge_index, edge_attr, W1, b1, g1, be1, W2, b2, g2, be2,
           Wl1, bl1, g3, be3, Wl2, bl2):
    row = edge_index[0].astype(jnp.int32)
    col = edge_index[1].astype(jnp.int32)
    ew = edge_attr.astype(jnp.float32)
    pad = E_PAD - E
    row_p = jnp.concatenate([row, jnp.zeros((pad,), jnp.int32)])
    col_p = jnp.concatenate([col, jnp.zeros((pad,), jnp.int32)])
    ew_p = jnp.concatenate([ew, jnp.zeros((pad,), jnp.float32)])
    row3 = row_p.reshape(NW, CPT, CHUNK)
    col3 = col_p.reshape(NW, CPT, CHUNK)
    ew3 = ew_p.reshape(NW, CPT, CHUNK)

    degp = _deg_kernel(col3, ew3)

    W1p = W1[:, _PERM]
    W2p = W2[:, _PERM]
    dinv, y1, y1m = _tc_call(
        _pre_body,
        (jax.ShapeDtypeStruct((N, 1), jnp.float32),
         jax.ShapeDtypeStruct((N, H), jnp.float32),
         jax.ShapeDtypeStruct((N, H), jnp.bfloat16)),
        x, W1, W1p, degp)

    agg1 = _agg_kernel(y1m, row3, col3, ew3)

    y2, y2m = _tc_call(
        _mid_body,
        (jax.ShapeDtypeStruct((N, H), jnp.float32),
         jax.ShapeDtypeStruct((N, H), jnp.bfloat16)),
        agg1, y1, dinv, W2, W2p,
        b1.reshape(1, H), g1.reshape(1, H), be1.reshape(1, H))

    agg2 = _agg_kernel(y2m, row3, col3, ew3)

    out = _tc_call(
        _post_body,
        jax.ShapeDtypeStruct((N, 1), jnp.float32),
        agg2, y2, dinv,
        b2.reshape(1, H), g2.reshape(1, H), be2.reshape(1, H),
        Wl1, bl1.reshape(1, H), g3.reshape(1, H), be3.reshape(1, H),
        Wl2, bl2.reshape(1, 1))

    return out.reshape(N)
